# Initial kernel scaffold; baseline (speedup 1.0000x reference)
import jax
import jax.numpy as jnp
from jax.experimental import pallas as pl


def kernel(x, embedding):
    raise NotImplementedError("write your pallas kernel here")


# speedup vs baseline: 4.8525x; 4.8525x over previous
import functools

import jax
import jax.numpy as jnp
from jax import lax
from jax.experimental import pallas as pl
from jax.experimental.pallas import tpu as pltpu, tpu_sc as plsc

_D = 128
_C = 128


@functools.lru_cache(maxsize=None)
def _make_gather(total):
    info = plsc.get_sparse_core_info()
    nc, ns = info.num_cores, info.num_subcores
    nw = nc * ns
    b_per_w = total // nw
    n_chunks = b_per_w // _C
    mesh = plsc.VectorSubcoreMesh(core_axis_name="c", subcore_axis_name="s")

    @functools.partial(
        pl.kernel,
        mesh=mesh,
        out_type=jax.ShapeDtypeStruct((total, _D), jnp.float32),
        scratch_types=[
            pltpu.VMEM((_C,), jnp.int32),
            pltpu.VMEM((_C, _D), jnp.float32),
            pltpu.SemaphoreType.DMA,
        ],
    )
    def gather_kernel(idx_hbm, table_hbm, out_hbm, idx_v, rows_v, sem):
        wid = lax.axis_index("s") * nc + lax.axis_index("c")
        base = wid * b_per_w

        def body(i, carry):
            off = base + i * _C
            pltpu.sync_copy(idx_hbm.at[pl.ds(off, _C)], idx_v)
            pltpu.async_copy(table_hbm.at[idx_v], rows_v, sem).wait()
            pltpu.sync_copy(rows_v, out_hbm.at[pl.ds(off, _C)])
            return carry

        lax.fori_loop(0, n_chunks, body, 0)

    return gather_kernel


@jax.jit
def kernel(x, embedding):
    b, s = x.shape
    total = b * s
    flat = x.reshape(total)
    out = _make_gather(total)(flat, embedding)
    return out.reshape(b, s, _D)


# speedup vs baseline: 6.5884x; 1.3577x over previous
import functools

import jax
import jax.numpy as jnp
from jax import lax
from jax.experimental import pallas as pl
from jax.experimental.pallas import tpu as pltpu, tpu_sc as plsc

_D = 128
_C = 128


@functools.lru_cache(maxsize=None)
def _make_gather(total):
    info = plsc.get_sparse_core_info()
    nc, ns = info.num_cores, info.num_subcores
    nw = nc * ns
    b_per_w = total // nw
    n_chunks = b_per_w // _C
    assert n_chunks % 2 == 0 and n_chunks >= 4
    mesh = plsc.VectorSubcoreMesh(core_axis_name="c", subcore_axis_name="s")

    @functools.partial(
        pl.kernel,
        mesh=mesh,
        out_type=jax.ShapeDtypeStruct((total, _D), jnp.float32),
        scratch_types=[
            pltpu.VMEM((n_chunks, _C), jnp.int32),
            pltpu.VMEM((_C, _D), jnp.float32),
            pltpu.VMEM((_C, _D), jnp.float32),
            pltpu.SemaphoreType.DMA,
            pltpu.SemaphoreType.DMA,
            pltpu.SemaphoreType.DMA,
            pltpu.SemaphoreType.DMA,
        ],
    )
    def gather_kernel(idx_hbm, table_hbm, out_hbm, idx_v, buf_a, buf_b,
                      gs_a, gs_b, ss_a, ss_b):
        wid = lax.axis_index("s") * nc + lax.axis_index("c")
        base = wid * b_per_w
        pltpu.sync_copy(idx_hbm.at[wid], idx_v)

        def start_gather(i, buf, sem):
            pltpu.async_copy(table_hbm.at[idx_v.at[i]], buf, sem)

        def wait_gather(buf, sem):
            pltpu.make_async_copy(table_hbm.at[pl.ds(0, _C)], buf, sem).wait()

        def start_store(i, buf, sem):
            pltpu.async_copy(buf, out_hbm.at[pl.ds(base + i * _C, _C)], sem)

        def wait_store(buf, sem):
            pltpu.make_async_copy(buf, out_hbm.at[pl.ds(base, _C)], sem).wait()

        start_gather(0, buf_a, gs_a)
        wait_gather(buf_a, gs_a)
        start_store(0, buf_a, ss_a)
        start_gather(1, buf_b, gs_b)

        ring = (
            (buf_b, gs_b, ss_b, buf_a, gs_a, ss_a),
            (buf_a, gs_a, ss_a, buf_b, gs_b, ss_b),
        )

        def body(t, carry):
            for j in range(2):
                i = 1 + 2 * t + j
                buf, gs, ss, obuf, ogs, oss = ring[j]
                wait_gather(buf, gs)
                start_store(i, buf, ss)
                wait_store(obuf, oss)
                start_gather(i + 1, obuf, ogs)
            return carry

        lax.fori_loop(0, (n_chunks - 2) // 2, body, 0)

        wait_gather(buf_b, gs_b)
        start_store(n_chunks - 1, buf_b, ss_b)
        wait_store(buf_a, ss_a)
        wait_store(buf_b, ss_b)

    return gather_kernel


@jax.jit
def kernel(x, embedding):
    b, s = x.shape
    total = b * s
    flat = x.reshape(32, total // (32 * _C), _C)
    out = _make_gather(total)(flat, embedding)
    return out.reshape(b, s, _D)


# speedup vs baseline: 8.0053x; 1.2151x over previous
import functools

import jax
import jax.numpy as jnp
from jax import lax
from jax.experimental import pallas as pl
from jax.experimental.pallas import tpu as pltpu, tpu_sc as plsc

_D = 128
_C = 128
_NBUF = 4


@functools.lru_cache(maxsize=None)
def _make_gather(total):
    info = plsc.get_sparse_core_info()
    nc, ns = info.num_cores, info.num_subcores
    nw = nc * ns
    b_per_w = total // nw
    n_chunks = b_per_w // _C
    assert n_chunks >= 2 * _NBUF and (n_chunks - 1 - _NBUF) % _NBUF == 1
    mesh = plsc.VectorSubcoreMesh(core_axis_name="c", subcore_axis_name="s")

    @functools.partial(
        pl.kernel,
        mesh=mesh,
        out_type=jax.ShapeDtypeStruct((total, _D), jnp.float32),
        scratch_types=[
            pltpu.VMEM((n_chunks, _C), jnp.int32),
        ]
        + [pltpu.VMEM((_C, _D), jnp.float32) for _ in range(_NBUF)]
        + [pltpu.SemaphoreType.DMA for _ in range(2 * _NBUF)],
    )
    def gather_kernel(idx_hbm, table_hbm, out_hbm, idx_v, *rest):
        bufs = rest[:_NBUF]
        gs = rest[_NBUF:2 * _NBUF]
        ss = rest[2 * _NBUF:]
        wid = lax.axis_index("s") * nc + lax.axis_index("c")
        base = wid * b_per_w
        pltpu.sync_copy(idx_hbm.at[wid], idx_v)

        def start_gather(i, b):
            pltpu.async_copy(table_hbm.at[idx_v.at[i]], bufs[b], gs[b])

        def wait_gather(b):
            pltpu.make_async_copy(
                table_hbm.at[pl.ds(0, _C)], bufs[b], gs[b]).wait()

        def start_store(i, b):
            pltpu.async_copy(bufs[b], out_hbm.at[pl.ds(base + i * _C, _C)],
                             ss[b])

        def wait_store(b):
            pltpu.make_async_copy(bufs[b], out_hbm.at[pl.ds(base, _C)],
                                  ss[b]).wait()

        def step(i, b, store_wait=True, lookahead=True):
            wait_gather(b)
            start_store(i, b)
            if lookahead:
                pb = (b - 1) % _NBUF
                if store_wait:
                    wait_store(pb)
                start_gather(i + _NBUF - 1, pb)

        for b in range(_NBUF - 1):
            start_gather(b, b)
        step(0, 0, store_wait=False)

        def body(t, carry):
            for j in range(_NBUF):
                i = 1 + _NBUF * t + j
                step(i, (1 + j) % _NBUF)
            return carry

        n_main = (n_chunks - 1 - _NBUF) // _NBUF
        lax.fori_loop(0, n_main, body, 0)

        for i in range(1 + _NBUF * n_main, n_chunks):
            step(i, i % _NBUF, lookahead=(i + _NBUF - 1 < n_chunks))
        for i in range(n_chunks - _NBUF, n_chunks):
            wait_store(i % _NBUF)

    return gather_kernel


@jax.jit
def kernel(x, embedding):
    b, s = x.shape
    total = b * s
    flat = x.reshape(32, total // (32 * _C), _C)
    out = _make_gather(total)(flat, embedding)
    return out.reshape(b, s, _D)


# speedup vs baseline: 8.0656x; 1.0075x over previous
import functools

import jax
import jax.numpy as jnp
from jax import lax
from jax.experimental import pallas as pl
from jax.experimental.pallas import tpu as pltpu, tpu_sc as plsc

_D = 128
_C = 128
_NBUF = 6


@functools.lru_cache(maxsize=None)
def _make_gather(total):
    info = plsc.get_sparse_core_info()
    nc, ns = info.num_cores, info.num_subcores
    nw = nc * ns
    b_per_w = total // nw
    n_chunks = b_per_w // _C
    assert n_chunks >= 2 * _NBUF
    mesh = plsc.VectorSubcoreMesh(core_axis_name="c", subcore_axis_name="s")

    @functools.partial(
        pl.kernel,
        mesh=mesh,
        out_type=jax.ShapeDtypeStruct((total, _D), jnp.float32),
        scratch_types=[
            pltpu.VMEM((n_chunks, _C), jnp.int32),
        ]
        + [pltpu.VMEM((_C, _D), jnp.float32) for _ in range(_NBUF)]
        + [pltpu.SemaphoreType.DMA for _ in range(2 * _NBUF)],
    )
    def gather_kernel(idx_hbm, table_hbm, out_hbm, idx_v, *rest):
        bufs = rest[:_NBUF]
        gs = rest[_NBUF:2 * _NBUF]
        ss = rest[2 * _NBUF:]
        wid = lax.axis_index("s") * nc + lax.axis_index("c")
        base = wid * b_per_w
        pltpu.sync_copy(idx_hbm.at[wid], idx_v)

        def start_gather(i, b):
            pltpu.async_copy(table_hbm.at[idx_v.at[i]], bufs[b], gs[b])

        def wait_gather(b):
            pltpu.make_async_copy(
                table_hbm.at[pl.ds(0, _C)], bufs[b], gs[b]).wait()

        def start_store(i, b):
            pltpu.async_copy(bufs[b], out_hbm.at[pl.ds(base + i * _C, _C)],
                             ss[b])

        def wait_store(b):
            pltpu.make_async_copy(bufs[b], out_hbm.at[pl.ds(base, _C)],
                                  ss[b]).wait()

        def step(i, b, store_wait=True, lookahead=True):
            wait_gather(b)
            start_store(i, b)
            if lookahead:
                pb = (b - 1) % _NBUF
                if store_wait:
                    wait_store(pb)
                start_gather(i + _NBUF - 1, pb)

        for b in range(_NBUF - 1):
            start_gather(b, b)
        step(0, 0, store_wait=False)

        def body(t, carry):
            for j in range(_NBUF):
                i = 1 + _NBUF * t + j
                step(i, (1 + j) % _NBUF)
            return carry

        n_main = (n_chunks - 1 - _NBUF) // _NBUF
        lax.fori_loop(0, n_main, body, 0)

        for i in range(1 + _NBUF * n_main, n_chunks):
            step(i, i % _NBUF, lookahead=(i + _NBUF - 1 < n_chunks))
        for i in range(n_chunks - _NBUF, n_chunks):
            wait_store(i % _NBUF)

    return gather_kernel


@jax.jit
def kernel(x, embedding):
    b, s = x.shape
    total = b * s
    flat = x.reshape(32, total // (32 * _C), _C)
    out = _make_gather(total)(flat, embedding)
    return out.reshape(b, s, _D)


# speedup vs baseline: 8.1129x; 1.0059x over previous
import functools

import jax
import jax.numpy as jnp
from jax import lax
from jax.experimental import pallas as pl
from jax.experimental.pallas import tpu as pltpu, tpu_sc as plsc

_D = 128
_C = 128
_NBUF = 7


@functools.lru_cache(maxsize=None)
def _make_gather(total):
    info = plsc.get_sparse_core_info()
    nc, ns = info.num_cores, info.num_subcores
    nw = nc * ns
    b_per_w = total // nw
    n_chunks = b_per_w // _C
    assert n_chunks >= 2 * _NBUF
    mesh = plsc.VectorSubcoreMesh(core_axis_name="c", subcore_axis_name="s")

    @functools.partial(
        pl.kernel,
        mesh=mesh,
        out_type=jax.ShapeDtypeStruct((total, _D), jnp.float32),
        scratch_types=[
            pltpu.VMEM((n_chunks, _C), jnp.int32),
        ]
        + [pltpu.VMEM((_C, _D), jnp.float32) for _ in range(_NBUF)]
        + [pltpu.SemaphoreType.DMA for _ in range(2 * _NBUF)],
    )
    def gather_kernel(idx_hbm, table_hbm, out_hbm, idx_v, *rest):
        bufs = rest[:_NBUF]
        gs = rest[_NBUF:2 * _NBUF]
        ss = rest[2 * _NBUF:]
        wid = lax.axis_index("s") * nc + lax.axis_index("c")
        base = wid * b_per_w
        pltpu.sync_copy(idx_hbm.at[wid], idx_v)

        def start_gather(i, b):
            pltpu.async_copy(table_hbm.at[idx_v.at[i]], bufs[b], gs[b])

        def wait_gather(b):
            pltpu.make_async_copy(
                table_hbm.at[pl.ds(0, _C)], bufs[b], gs[b]).wait()

        def start_store(i, b):
            pltpu.async_copy(bufs[b], out_hbm.at[pl.ds(base + i * _C, _C)],
                             ss[b])

        def wait_store(b):
            pltpu.make_async_copy(bufs[b], out_hbm.at[pl.ds(base, _C)],
                                  ss[b]).wait()

        def step(i, b, store_wait=True, lookahead=True):
            wait_gather(b)
            start_store(i, b)
            if lookahead:
                pb = (b - 1) % _NBUF
                if store_wait:
                    wait_store(pb)
                start_gather(i + _NBUF - 1, pb)

        for b in range(_NBUF - 1):
            start_gather(b, b)
        step(0, 0, store_wait=False)

        def body(t, carry):
            for j in range(_NBUF):
                i = 1 + _NBUF * t + j
                step(i, (1 + j) % _NBUF)
            return carry

        n_main = (n_chunks - 1 - _NBUF) // _NBUF
        lax.fori_loop(0, n_main, body, 0)

        for i in range(1 + _NBUF * n_main, n_chunks):
            step(i, i % _NBUF, lookahead=(i + _NBUF - 1 < n_chunks))
        for i in range(n_chunks - _NBUF, n_chunks):
            wait_store(i % _NBUF)

    return gather_kernel


@jax.jit
def kernel(x, embedding):
    b, s = x.shape
    total = b * s
    flat = x.reshape(32, total // (32 * _C), _C)
    out = _make_gather(total)(flat, embedding)
    return out.reshape(b, s, _D)
